# Initial kernel scaffold; baseline (speedup 1.0000x reference)
import jax
import jax.numpy as jnp
from jax.experimental import pallas as pl


def kernel(x, edge_index, batch, emb_W, emb_b, U, bu, Wsp, bsp, V, bv, pWs, pWn, W1, b1, W2, b2, W3, b3):
    raise NotImplementedError("write your pallas kernel here")


# speedup vs baseline: 8.3758x; 8.3758x over previous
import functools

import jax
import jax.numpy as jnp
from jax import lax
from jax.experimental import pallas as pl
from jax.experimental.pallas import tpu as pltpu
from jax.experimental.pallas import tpu_sc as plsc

N = 10000
NP = 10240
E = 320000
B = 16
DF = 128
H = 128
FFN = 256
NC = 10

_SC_CORES = 2
_SC_TILES = 16
_NW = _SC_CORES * _SC_TILES
_EPW = E // _NW
_CH = 80
_NCHUNK = _EPW // _CH
_ZCH = NP // (_SC_TILES * _CH)

@functools.cache
def _build_sc_segsum():
    mesh = plsc.VectorSubcoreMesh(core_axis_name="c", subcore_axis_name="s")

    @functools.partial(
        pl.kernel,
        out_type=jax.ShapeDtypeStruct((_SC_CORES, NP, H), jnp.float32),
        mesh=mesh,
        scratch_types=[
            pltpu.VMEM((_NCHUNK, _CH), jnp.int32),
            pltpu.VMEM((_NCHUNK, _CH), jnp.int32),
            pltpu.VMEM((_CH, H), jnp.float32),
            pltpu.VMEM_SHARED((NP, H), jnp.float32),
            pltpu.SemaphoreType.DMA,
        ],
    )
    def _sc_segsum(table, src, dst, zrows, out, sidx, didx, rows, acc, sem):
        c = lax.axis_index("c")
        s = lax.axis_index("s")
        wid = s * _SC_CORES + c

        for k in range(_ZCH):
            row0 = (s * _ZCH + k) * _CH
            pltpu.sync_copy(zrows, acc.at[pl.ds(row0, _CH)])

        pltpu.sync_copy(src.at[wid], sidx)
        pltpu.sync_copy(dst.at[wid], didx)

        plsc.subcore_barrier()

        def step(j, carry):
            pltpu.async_copy(table.at[sidx.at[j]], rows, sem).wait()
            pltpu.sync_copy(rows, acc.at[didx.at[j]], add=True)
            return carry

        lax.fori_loop(0, _NCHUNK, step, 0)

        plsc.subcore_barrier()

        for k in range(_ZCH):
            row0 = (s * _ZCH + k) * _CH
            pltpu.sync_copy(acc.at[pl.ds(row0, _CH)], out.at[c, pl.ds(row0, _CH)])

    return _sc_segsum


def _segsum(table, src, dst, zrows):
    return _build_sc_segsum()(table, src, dst, zrows)


def _dot(a, b):
    return jnp.dot(a.astype(jnp.bfloat16), b.astype(jnp.bfloat16),
                   preferred_element_type=jnp.float32)


def _lnf(h):
    m = jnp.mean(h, axis=-1, keepdims=True)
    v = jnp.var(h, axis=-1, keepdims=True)
    return (h - m) / jnp.sqrt(v + 1e-5)


def _emb_body(x_ref, w_ref, b_ref, h_ref):
    h_ref[...] = (
        _dot(x_ref[...], w_ref[...])
        + b_ref[...]
    )


def _pre_body(h_ref, u_ref, bu_ref, nm_ref, z1_ref, z2m_ref):
    h = h_ref[...]
    z = jax.nn.gelu(
        _dot(_lnf(h), u_ref[...])
        + bu_ref[...]
    )
    z1_ref[...] = z[:, :H]
    z2m_ref[...] = _lnf(z[:, H:]) * nm_ref[...]


def _post_body(p_ref, z1_ref, wsp_ref, bsp_ref, v_ref, bv_ref, h_ref, nm_ref,
               ys_ref, hn_ref, ysn_ref):
    agg = (p_ref[0] + p_ref[1]) * nm_ref[...]
    gate = _dot(agg, wsp_ref[...]) + bsp_ref[...]
    hn = (
        _dot(z1_ref[...] * gate, v_ref[...])
        + bv_ref[...]
        + h_ref[...]
    )
    hn_ref[...] = hn
    ysn_ref[...] = ys_ref[...] + hn


def _readout_body(ys_ref, bcol_ref, nm_ref, r_ref):
    y = ys_ref[...]
    bcol = bcol_ref[...]
    nm = nm_ref[...]
    ym = y * nm
    for g in range(B):
        mg = bcol == g
        cnt = jnp.sum(jnp.where(mg, nm, 0.0))
        sg = jnp.sum(jnp.where(mg, ym, 0.0), axis=0)
        mx = jnp.max(jnp.where(mg & (nm > 0.0), y, -1e9), axis=0)
        r_ref[g, :H] = sg / jnp.maximum(cnt, 1.0)
        r_ref[g, H:] = jnp.where(cnt > 0.0, mx, 0.0)


def _score_body(p_ref, h_ref, pws_ref, pwn_ref, s_ref):
    agg = p_ref[0] + p_ref[1]
    s_ref[...] = (
        _dot(h_ref[...], pws_ref[...])
        + _dot(agg, pwn_ref[...])
    )


def _topk_body(s2_ref, b2_ref, nm2_ref):
    b2 = b2_ref[...]
    u = jax.lax.bitcast_convert_type(s2_ref[...], jnp.int32)
    sign = jnp.int32(-2147483648)
    key = jnp.where(u < 0, jnp.bitwise_not(u) ^ sign, u)

    ks = [
        (jnp.sum(jnp.where(b2 == g, 1, 0).astype(jnp.int32)) + 1) // 2
        for g in range(B)
    ]

    def it(t, carry):
        los, his = carry[:B], carry[B:]
        nlo, nhi = [], []
        for g in range(B):
            lo, hi = los[g], his[g]
            mid = (lo >> 1) + (hi >> 1) + (lo & hi & 1) + ((lo ^ hi) & 1)
            cnt = jnp.sum(
                jnp.where((b2 == g) & (key >= mid), 1, 0).astype(jnp.int32))
            ok = cnt >= ks[g]
            nlo.append(jnp.where(ok, mid, lo))
            nhi.append(jnp.where(ok, hi, mid - 1))
        return tuple(nlo) + tuple(nhi)

    init = tuple([jnp.int32(-2147483648)] * B) + tuple([jnp.int32(2147483647)] * B)
    res = lax.fori_loop(0, 32, it, init)

    thr = jnp.full(b2.shape, jnp.int32(2147483647), jnp.int32)
    for g in range(B):
        thr = jnp.where(b2 == g, res[g], thr)
    nm2_ref[...] = ((key >= thr) & (b2 < B)).astype(jnp.float32)


def _apply_body(h_ref, s_ref, nm_ref, hn_ref):
    hn_ref[...] = h_ref[...] * jnp.tanh(s_ref[...]) * nm_ref[...]


def _final_body(r1_ref, r2_ref, w1_ref, b1_ref, w2_ref, b2_ref, w3_ref, b3_ref,
                o_ref):
    g = r1_ref[...] + r2_ref[...]
    g = jax.nn.relu(_dot(g, w1_ref[...]) + b1_ref[...])
    g = jax.nn.relu(_dot(g, w2_ref[...]) + b2_ref[...])
    o_ref[...] = _dot(g, w3_ref[...]) + b3_ref[...]


def _tc(body, out_shapes):
    return pl.pallas_call(body, out_shape=out_shapes)


_RB = 1280
_NRB = NP // _RB


def _rows(shape):
    return pl.BlockSpec((_RB, shape), lambda i: (i, 0))


def _full(*shape):
    n = len(shape)
    return pl.BlockSpec(shape, lambda i: (0,) * n)


def _emb_call(xp, w, b):
    return pl.pallas_call(
        _emb_body,
        grid=(_NRB,),
        in_specs=[_rows(DF), _full(DF, H), _full(1, H)],
        out_specs=_rows(H),
        out_shape=jax.ShapeDtypeStruct((NP, H), jnp.float32),
    )(xp, w, b)


def _pre_call(h, u, bu, nm):
    return pl.pallas_call(
        _pre_body,
        grid=(_NRB,),
        in_specs=[_rows(H), _full(H, FFN), _full(1, FFN), _rows(1)],
        out_specs=(_rows(H), _rows(H)),
        out_shape=(jax.ShapeDtypeStruct((NP, H), jnp.float32),
                   jax.ShapeDtypeStruct((NP, H), jnp.float32)),
    )(h, u, bu, nm)


def _post_call(p, z1, wsp, bsp, v, bv, h, nm, ys):
    return pl.pallas_call(
        _post_body,
        grid=(_NRB,),
        in_specs=[pl.BlockSpec((2, _RB, H), lambda i: (0, i, 0)),
                  _rows(H), _full(H, H), _full(1, H), _full(H, H),
                  _full(1, H), _rows(H), _rows(1), _rows(H)],
        out_specs=(_rows(H), _rows(H)),
        out_shape=(jax.ShapeDtypeStruct((NP, H), jnp.float32),
                   jax.ShapeDtypeStruct((NP, H), jnp.float32)),
    )(p, z1, wsp, bsp, v, bv, h, nm, ys)


def kernel(x, edge_index, batch, emb_W, emb_b, U, bu, Wsp, bsp, V, bv,
           pWs, pWn, W1, b1, W2, b2, W3, b3):
    f32 = jnp.float32
    xp = jnp.pad(x, ((0, NP - N), (0, 0)))
    bcol = jnp.pad(batch.astype(jnp.int32), (0, NP - N), constant_values=B)
    bcol = bcol.reshape(NP, 1)
    src = edge_index[0].astype(jnp.int32).reshape(_NW, _NCHUNK, _CH)
    dst = edge_index[1].astype(jnp.int32).reshape(_NW, _NCHUNK, _CH)
    zrows = jnp.zeros((_CH, H), f32)
    ones_col = jnp.ones((NP, 1), f32)
    zero_h = jnp.zeros((NP, H), f32)

    fA = jax.ShapeDtypeStruct((NP, H), f32)
    fC = jax.ShapeDtypeStruct((NP, 1), f32)
    fR = jax.ShapeDtypeStruct((B, 2 * H), f32)

    h = _emb_call(xp, emb_W, emb_b.reshape(1, H))

    def sublayer(h, ys, i, nm):
        z1, z2m = _pre_call(h, U[i], bu[i].reshape(1, FFN), nm)
        p = _segsum(z2m, src, dst, zrows)
        return _post_call(p, z1, Wsp[i], bsp[i].reshape(1, FFN // 2), V[i],
                          bv[i].reshape(1, H), h, nm, ys)

    h, ys = sublayer(h, zero_h, 0, ones_col)
    h, ys = sublayer(h, ys, 1, ones_col)
    r1 = _tc(_readout_body, fR)(ys, bcol, ones_col)

    p = _segsum(h, src, dst, zrows)
    score = pl.pallas_call(
        _score_body,
        grid=(_NRB,),
        in_specs=[pl.BlockSpec((2, _RB, H), lambda i: (0, i, 0)),
                  _rows(H), _full(H, 1), _full(H, 1)],
        out_specs=_rows(1),
        out_shape=fC,
    )(p, h, pWs, pWn)
    nm2 = pl.pallas_call(
        _topk_body,
        out_shape=jax.ShapeDtypeStruct((NP // 128, 128), f32),
    )(score.reshape(NP // 128, 128), bcol.reshape(NP // 128, 128))
    nm = nm2.reshape(NP, 1)
    h = pl.pallas_call(
        _apply_body,
        grid=(_NRB,),
        in_specs=[_rows(H), _rows(1), _rows(1)],
        out_specs=_rows(H),
        out_shape=fA,
    )(h, score, nm)

    h, ys = sublayer(h, zero_h, 2, nm)
    h, ys = sublayer(h, ys, 3, nm)
    r2 = _tc(_readout_body, fR)(ys, bcol, nm)

    return _tc(_final_body, jax.ShapeDtypeStruct((B, NC), f32))(
        r1, r2, W1, b1.reshape(1, H), W2, b2.reshape(1, H),
        W3, b3.reshape(1, NC))
